# baseline (speedup 1.0000x reference)
import jax
import jax.numpy as jnp
from jax import lax
from jax.experimental import pallas as pl
from jax.experimental.pallas import tpu as pltpu
from jax.experimental.pallas import tpu_sc as plsc

_WINDOW = 80
_SPLIT = 256
_TAIL = 128
_TBLK = 800


def _tail_table(table, dim):
    vocab = table.shape[0]

    def body(t_ref, o_ref):
        tail = t_ref[:, _SPLIT:dim]
        o_ref[...] = jnp.concatenate(
            [tail, jnp.zeros((_TBLK, _TAIL - (dim - _SPLIT)), jnp.float32)], axis=1
        )

    return pl.pallas_call(
        body,
        grid=(vocab // _TBLK,),
        in_specs=[pl.BlockSpec((_TBLK, table.shape[1]), lambda i: (i, 0))],
        out_specs=pl.BlockSpec((_TBLK, _TAIL), lambda i: (i, 0)),
        out_shape=jax.ShapeDtypeStruct((vocab, _TAIL), jnp.float32),
    )(table)


def kernel(sentence, table):
    batch, seq = sentence.shape
    vocab, dim = table.shape
    n = batch * seq
    idx = sentence.reshape(n).astype(jnp.int32)
    tail_tab = _tail_table(table, dim)

    info = plsc.get_sparse_core_info()
    nw = info.num_cores * info.num_subcores
    per_w = n // nw
    steps = per_w // _WINDOW
    assert steps % 2 == 0

    mesh = plsc.VectorSubcoreMesh(core_axis_name="core", subcore_axis_name="subcore")

    @pl.kernel(
        out_type=jax.ShapeDtypeStruct((n, dim), table.dtype),
        mesh=mesh,
        scratch_types=[
            pltpu.VMEM((per_w,), jnp.int32),
            pltpu.VMEM((_WINDOW, dim), jnp.float32),
            pltpu.VMEM((_WINDOW, dim), jnp.float32),
            pltpu.VMEM((_WINDOW, _TAIL), jnp.float32),
            pltpu.VMEM((_WINDOW, _TAIL), jnp.float32),
            pltpu.SemaphoreType.DMA,
            pltpu.SemaphoreType.DMA,
        ],
    )
    def gather_kernel(
        tab_hbm, tail_hbm, idx_hbm, out_hbm, idx_v, rows_a, rows_b, t_a, t_b, sem_a, sem_b
    ):
        wid = lax.axis_index("subcore") * info.num_cores + lax.axis_index("core")
        base = wid * per_w
        pltpu.sync_copy(idx_hbm.at[pl.ds(base, per_w)], idx_v)

        def issue(w, rows, tv, sem):
            iv = idx_v.at[pl.ds(w * _WINDOW, _WINDOW)]
            pltpu.async_copy(tab_hbm.at[:, pl.ds(0, _SPLIT)].at[iv], rows.at[:, pl.ds(0, _SPLIT)], sem)
            pltpu.async_copy(tail_hbm.at[iv], tv, sem)

        def finish(w, rows, tv, sem):
            pltpu.make_async_copy(tab_hbm.at[pl.ds(0, _WINDOW), pl.ds(0, _SPLIT)], rows.at[:, pl.ds(0, _SPLIT)], sem).wait()
            pltpu.make_async_copy(tail_hbm.at[pl.ds(0, _WINDOW)], tv, sem).wait()

            @pl.loop(0, _WINDOW)
            def _(j):
                rows[j, pl.ds(_SPLIT, 16)] = tv[j, pl.ds(0, 16)]
                rows[j, pl.ds(_SPLIT + 16, 16)] = tv[j, pl.ds(16, 16)]
                rows[j, pl.ds(_SPLIT + 32, 12)] = tv[j, pl.ds(32, 12)]

            pltpu.sync_copy(rows, out_hbm.at[pl.ds(base + w * _WINDOW, _WINDOW)])

        @pl.loop(0, steps, step=2)
        def _(w):
            issue(w, rows_a, t_a, sem_a)
            issue(w + 1, rows_b, t_b, sem_b)
            finish(w, rows_a, t_a, sem_a)
            finish(w + 1, rows_b, t_b, sem_b)

    out = gather_kernel(table, tail_tab, idx)
    return out.reshape(batch, seq, dim)


# speedup vs baseline: 1.2134x; 1.2134x over previous
import jax
import jax.numpy as jnp
from jax import lax
from jax.experimental import pallas as pl
from jax.experimental.pallas import tpu as pltpu
from jax.experimental.pallas import tpu_sc as plsc

_SPLIT = 256
_TAIL = 128
_TBLK = 800


def _tail_table(table, dim):
    vocab = table.shape[0]

    def body(t_ref, o_ref):
        tail = t_ref[:, _SPLIT:dim]
        o_ref[...] = jnp.concatenate(
            [tail, jnp.zeros((_TBLK, _TAIL - (dim - _SPLIT)), jnp.float32)], axis=1
        )

    return pl.pallas_call(
        body,
        grid=(vocab // _TBLK,),
        in_specs=[pl.BlockSpec((_TBLK, table.shape[1]), lambda i: (i, 0))],
        out_specs=pl.BlockSpec((_TBLK, _TAIL), lambda i: (i, 0)),
        out_shape=jax.ShapeDtypeStruct((vocab, _TAIL), jnp.float32),
    )(table)


def kernel(sentence, table):
    batch, seq = sentence.shape
    vocab, dim = table.shape
    sent = sentence.astype(jnp.int32)
    tail_tab = _tail_table(table, dim)

    info = plsc.get_sparse_core_info()
    nw = info.num_cores * info.num_subcores
    per_w = batch // nw
    assert per_w % 2 == 0

    mesh = plsc.VectorSubcoreMesh(core_axis_name="core", subcore_axis_name="subcore")

    @pl.kernel(
        out_type=jax.ShapeDtypeStruct((batch, seq, dim), table.dtype),
        mesh=mesh,
        scratch_types=[
            pltpu.VMEM((per_w, seq), jnp.int32),
            pltpu.VMEM((seq, dim), jnp.float32),
            pltpu.VMEM((seq, dim), jnp.float32),
            pltpu.VMEM((seq, _TAIL), jnp.float32),
            pltpu.VMEM((seq, _TAIL), jnp.float32),
            pltpu.SemaphoreType.DMA,
            pltpu.SemaphoreType.DMA,
        ],
    )
    def gather_kernel(
        tab_hbm, tail_hbm, idx_hbm, out_hbm, idx_v, rows_a, rows_b, t_a, t_b, sem_a, sem_b
    ):
        wid = lax.axis_index("subcore") * info.num_cores + lax.axis_index("core")
        base = wid * per_w
        pltpu.sync_copy(idx_hbm.at[pl.ds(base, per_w)], idx_v)

        def issue(c, rows, tv, sem):
            iv = idx_v.at[c]
            h1 = pltpu.async_copy(
                tab_hbm.at[:, pl.ds(0, _SPLIT)].at[iv], rows.at[:, pl.ds(0, _SPLIT)], sem
            )
            h2 = pltpu.async_copy(tail_hbm.at[iv], tv, sem)
            return h1, h2

        def finish(c, rows, tv, handles):
            for h in handles:
                h.wait()

            @pl.loop(0, seq)
            def _(j):
                rows[j, pl.ds(_SPLIT, 16)] = tv[j, pl.ds(0, 16)]
                rows[j, pl.ds(_SPLIT + 16, 16)] = tv[j, pl.ds(16, 16)]
                rows[j, pl.ds(_SPLIT + 32, 12)] = tv[j, pl.ds(32, 12)]

            pltpu.sync_copy(rows, out_hbm.at[base + c])

        @pl.loop(0, per_w, step=2)
        def _(c):
            ha = issue(c, rows_a, t_a, sem_a)
            hb = issue(c + 1, rows_b, t_b, sem_b)
            finish(c, rows_a, t_a, ha)
            finish(c + 1, rows_b, t_b, hb)

    return gather_kernel(table, tail_tab, sent)
